# baseline (speedup 1.0000x reference)
import functools

import jax
import jax.numpy as jnp
from jax import lax
from jax.experimental import pallas as pl
from jax.experimental.pallas import tpu as pltpu
from jax.experimental.pallas import tpu_sc as plsc

N = 10000
E = 320000
Q = 65536
D = 128
NP = 10240

NC = 2
NS = 16
NW = NC * NS

ECHUNK = 64
EPT = E // NW
EPT_PAD = 10240
NCH = EPT_PAD // ECHUNK
NBUF = 2
NG = NCH // NBUF

QCHUNK = 128
QNCH = 2 * (Q // NW) // QCHUNK
QPT = Q // NW

ROWS_PER_TILE = NP // NS

_MESH = plsc.VectorSubcoreMesh(core_axis_name="c", subcore_axis_name="s")


DEGW = 128
DEG_FD = 8


@functools.partial(
    pl.kernel,
    out_type=jax.ShapeDtypeStruct((NC * NP, DEGW), jnp.float32),
    mesh=_MESH,
    scratch_types=[
        pltpu.VMEM((NCH, ECHUNK), jnp.int32),
        pltpu.VMEM((ECHUNK, DEGW), jnp.float32),
        pltpu.VMEM_SHARED((NP, DEGW), jnp.float32),
        pltpu.SemaphoreType.DMA,
    ],
)
def _deg_sc(dst_hbm, ones_hbm, zeros_hbm, out_hbm, dstv, onesv, deg, sem):
    cid = lax.axis_index("c")
    sid = lax.axis_index("s")
    wid = cid * NS + sid
    r0 = sid * ROWS_PER_TILE
    pltpu.sync_copy(zeros_hbm, deg.at[pl.ds(r0, ROWS_PER_TILE)])
    pltpu.sync_copy(ones_hbm, onesv)
    pltpu.sync_copy(dst_hbm.at[wid], dstv)
    plsc.subcore_barrier()

    def d_desc(j):
        return pltpu.make_async_copy(onesv, deg.at[dstv.at[j]], sem)

    def body(j, carry):
        d_desc(j).start(add=True)

        @pl.when(j >= DEG_FD)
        def _():
            d_desc(j - DEG_FD).wait()

        return carry

    lax.fori_loop(0, NCH, body, 0)
    for k in range(DEG_FD):
        d_desc(NCH - DEG_FD + k).wait()
    plsc.subcore_barrier()
    pltpu.sync_copy(deg.at[pl.ds(r0, ROWS_PER_TILE)],
                    out_hbm.at[pl.ds(cid * NP + r0, ROWS_PER_TILE)])


ECONV = 80
CNCH = EPT_PAD // ECONV


@functools.partial(
    pl.kernel,
    out_type=jax.ShapeDtypeStruct((NC * NP, D), jnp.float32),
    mesh=_MESH,
    scratch_types=[
        pltpu.VMEM((ECONV,), jnp.int32),
        pltpu.VMEM((ECONV,), jnp.int32),
        pltpu.VMEM((ECONV, D), jnp.float32),
        pltpu.VMEM_SHARED((NP, D), jnp.float32),
        pltpu.SemaphoreType.DMA,
    ],
)
def _conv_sc(h_hbm, src_hbm, dst_hbm, out_hbm, sbuf, dbuf, rows, acc, sem):
    cid = lax.axis_index("c")
    sid = lax.axis_index("s")
    wid = cid * NS + sid
    r0 = sid * ROWS_PER_TILE
    pltpu.sync_copy(h_hbm.at[pl.ds(r0, ROWS_PER_TILE)],
                    acc.at[pl.ds(r0, ROWS_PER_TILE)])
    plsc.subcore_barrier()

    def body(j, carry):
        pltpu.sync_copy(src_hbm.at[wid, j], sbuf)
        pltpu.sync_copy(dst_hbm.at[wid, j], dbuf)
        pltpu.async_copy(h_hbm.at[sbuf], rows, sem).wait()
        pltpu.sync_copy(rows, acc.at[dbuf], add=True)
        return carry

    lax.fori_loop(0, CNCH, body, 0)
    plsc.subcore_barrier()
    pltpu.sync_copy(acc.at[pl.ds(r0, ROWS_PER_TILE)],
                    out_hbm.at[pl.ds(cid * NP + r0, ROWS_PER_TILE)])


@functools.partial(
    pl.kernel,
    out_type=jax.ShapeDtypeStruct((2 * Q, D), jnp.float32),
    mesh=_MESH,
    scratch_types=[
        pltpu.VMEM((QNCH, QCHUNK), jnp.int32),
    ]
    + [pltpu.VMEM((QCHUNK, D), jnp.float32)] * NBUF
    + [pltpu.SemaphoreType.DMA] * (2 * NBUF),
)
def _link_sc(x_hbm, q_hbm, out_hbm, qv, *bufs):
    rows = bufs[:NBUF]
    gsem = bufs[NBUF:2 * NBUF]
    ssem = bufs[2 * NBUF:]
    cid = lax.axis_index("c")
    sid = lax.axis_index("s")
    wid = cid * NS + sid
    pltpu.sync_copy(q_hbm.at[wid], qv)

    half = QNCH // 2

    def out_off(j):
        sel = j // half
        return sel * Q + wid * QPT + (j - sel * half) * QCHUNK

    def g_desc(b, j):
        return pltpu.make_async_copy(x_hbm.at[qv.at[j]], rows[b], gsem[b])

    def s_desc(b, j):
        return pltpu.make_async_copy(
            rows[b], out_hbm.at[pl.ds(out_off(j), QCHUNK)], ssem[b])

    for b in range(NBUF):
        g_desc(b, b).start()

    def outer(o, carry):
        j0 = o * NBUF
        for b in range(NBUF):
            g_desc(b, j0 + b).wait()
            s_desc(b, j0 + b).start()
        for b in range(NBUF):
            s_desc(b, j0 + b).wait()
            nj = j0 + NBUF + b

            @pl.when(nj < QNCH)
            def _():
                g_desc(b, nj).start()

        return carry

    lax.fori_loop(0, QNCH // NBUF, outer, 0)


_RB = 1024


def _dis_body(degp_ref, dis_ref):
    d = degp_ref[...]
    deg = d[0:NP, 0:1] + d[NP:2 * NP, 0:1] + 1.0
    dis_ref[...] = lax.rsqrt(deg)


def _dis_tc(degp):
    return pl.pallas_call(
        _dis_body,
        grid=(1,),
        in_specs=[pl.BlockSpec((NC * NP, DEGW), lambda i: (0, 0))],
        out_specs=pl.BlockSpec((NP, 1), lambda i: (0, 0)),
        out_shape=jax.ShapeDtypeStruct((NP, 1), jnp.float32),
    )(degp)


def _prep_body(dis_ref, emb_ref, w1_ref, h_ref):
    h_ref[...] = jnp.dot(emb_ref[...] * dis_ref[...], w1_ref[...],
                         preferred_element_type=jnp.float32)


def _prep_tc(dis, emb, w1):
    return pl.pallas_call(
        _prep_body,
        grid=(NP // _RB,),
        in_specs=[
            pl.BlockSpec((_RB, 1), lambda i: (i, 0)),
            pl.BlockSpec((_RB, D), lambda i: (i, 0)),
            pl.BlockSpec((D, D), lambda i: (0, 0)),
        ],
        out_specs=pl.BlockSpec((_RB, D), lambda i: (i, 0)),
        out_shape=jax.ShapeDtypeStruct((NP, D), jnp.float32),
    )(dis, emb, w1)


def _mid_body(acca_ref, accb_ref, hp_ref, dis_ref, b_ref, w_ref, out_ref):
    s = acca_ref[...] + accb_ref[...] - hp_ref[...]
    x1 = jnp.maximum(dis_ref[...] * s + b_ref[...], 0.0)
    out_ref[...] = jnp.dot(x1 * dis_ref[...], w_ref[...],
                           preferred_element_type=jnp.float32)


def _mid_tc(acc, hp, dis, b_row, w2):
    return pl.pallas_call(
        _mid_body,
        grid=(NP // _RB,),
        in_specs=[
            pl.BlockSpec((_RB, D), lambda i: (i, 0)),
            pl.BlockSpec((_RB, D), lambda i: (i + NP // _RB, 0)),
            pl.BlockSpec((_RB, D), lambda i: (i, 0)),
            pl.BlockSpec((_RB, 1), lambda i: (i, 0)),
            pl.BlockSpec((1, D), lambda i: (0, 0)),
            pl.BlockSpec((D, D), lambda i: (0, 0)),
        ],
        out_specs=pl.BlockSpec((_RB, D), lambda i: (i, 0)),
        out_shape=jax.ShapeDtypeStruct((NP, D), jnp.float32),
    )(acc, acc, hp, dis, b_row, w2)


def _final_body(acca_ref, accb_ref, hp_ref, dis_ref, b_ref, out_ref):
    s = acca_ref[...] + accb_ref[...] - hp_ref[...]
    out_ref[...] = dis_ref[...] * s + b_ref[...]


def _final_tc(acc, hp, dis, b_row):
    return pl.pallas_call(
        _final_body,
        grid=(NP // _RB,),
        in_specs=[
            pl.BlockSpec((_RB, D), lambda i: (i, 0)),
            pl.BlockSpec((_RB, D), lambda i: (i + NP // _RB, 0)),
            pl.BlockSpec((_RB, D), lambda i: (i, 0)),
            pl.BlockSpec((_RB, 1), lambda i: (i, 0)),
            pl.BlockSpec((1, D), lambda i: (0, 0)),
        ],
        out_specs=pl.BlockSpec((_RB, D), lambda i: (i, 0)),
        out_shape=jax.ShapeDtypeStruct((NP, D), jnp.float32),
    )(acc, acc, hp, dis, b_row)


_QB = 2048


def _pred_body(ga_ref, gb_ref, wp1_ref, bp1_ref, wp2_ref, bp2_ref, out_ref):
    h = ga_ref[...] * gb_ref[...]
    h = jnp.maximum(
        jnp.dot(h, wp1_ref[...], preferred_element_type=jnp.float32)
        + bp1_ref[...], 0.0)
    z = jnp.dot(h, wp2_ref[...], preferred_element_type=jnp.float32) \
        + bp2_ref[...]
    out_ref[...] = jax.nn.sigmoid(z)


def _pred_tc(gab, wp1, bp1_row, wp2, bp2_row):
    return pl.pallas_call(
        _pred_body,
        grid=(Q // _QB,),
        in_specs=[
            pl.BlockSpec((_QB, D), lambda i: (i, 0)),
            pl.BlockSpec((_QB, D), lambda i: (i + Q // _QB, 0)),
            pl.BlockSpec((D, D), lambda i: (0, 0)),
            pl.BlockSpec((1, D), lambda i: (0, 0)),
            pl.BlockSpec((D, 1), lambda i: (0, 0)),
            pl.BlockSpec((1, 1), lambda i: (0, 0)),
        ],
        out_specs=pl.BlockSpec((_QB, 1), lambda i: (i, 0)),
        out_shape=jax.ShapeDtypeStruct((Q, 1), jnp.float32),
    )(gab, gab, wp1, bp1_row, wp2, bp2_row)


def kernel(edge_index, edges, emb, W1, b1, W2, b2, Wp1, bp1, Wp2, bp2):
    src = edge_index[0].astype(jnp.int32)
    dst = edge_index[1].astype(jnp.int32)
    pad = EPT_PAD - EPT
    src3 = jnp.pad(src.reshape(NW, EPT), ((0, 0), (0, pad))) \
        .reshape(NW, NCH, ECHUNK)
    dst3 = jnp.pad(dst.reshape(NW, EPT), ((0, 0), (0, pad)),
                   constant_values=NP - 1).reshape(NW, NCH, ECHUNK)
    srcc = src3.reshape(NW, CNCH, ECONV)
    dstc = dst3.reshape(NW, CNCH, ECONV)
    q0 = edges[0].astype(jnp.int32).reshape(NW, QNCH // 2, QCHUNK)
    q1 = edges[1].astype(jnp.int32).reshape(NW, QNCH // 2, QCHUNK)
    qall = jnp.concatenate([q0, q1], axis=1)

    emb_p = jnp.pad(emb, ((0, NP - N), (0, 0)))
    degp = _deg_sc(dst3,
                   jnp.ones((ECHUNK, DEGW), jnp.float32),
                   jnp.zeros((ROWS_PER_TILE, DEGW), jnp.float32))
    dis = _dis_tc(degp)
    h1p = _prep_tc(dis, emb_p, W1)
    acc1 = _conv_sc(h1p, srcc, dstc)
    h2p = _mid_tc(acc1, h1p, dis, b1.reshape(1, D), W2)
    acc2 = _conv_sc(h2p, srcc, dstc)
    x2 = _final_tc(acc2, h2p, dis, b2.reshape(1, D))
    gab = _link_sc(x2, qall)
    out = _pred_tc(gab, Wp1, bp1.reshape(1, D), Wp2, bp2.reshape(1, 1))
    return out[:, 0]


# speedup vs baseline: 1.5813x; 1.5813x over previous
import functools

import jax
import jax.numpy as jnp
from jax import lax
from jax.experimental import pallas as pl
from jax.experimental.pallas import tpu as pltpu
from jax.experimental.pallas import tpu_sc as plsc

N = 10000
E = 320000
Q = 65536
D = 128
NP = 10240

NC = 2
NS = 16
NW = NC * NS

ECHUNK = 64
EPT = E // NW
EPT_PAD = 10240
NCH = EPT_PAD // ECHUNK
NBUF = 2
NG = NCH // NBUF

QCHUNK = 128
QNCH = 2 * (Q // NW) // QCHUNK
QPT = Q // NW

ROWS_PER_TILE = NP // NS

_MESH = plsc.VectorSubcoreMesh(core_axis_name="c", subcore_axis_name="s")


DEGW = 128
DEG_FD = 8


@functools.partial(
    pl.kernel,
    out_type=jax.ShapeDtypeStruct((NC * NP, DEGW), jnp.float32),
    mesh=_MESH,
    scratch_types=[
        pltpu.VMEM((NCH, ECHUNK), jnp.int32),
        pltpu.VMEM((ECHUNK, DEGW), jnp.float32),
        pltpu.VMEM_SHARED((NP, DEGW), jnp.float32),
        pltpu.SemaphoreType.DMA,
    ],
)
def _deg_sc(dst_hbm, ones_hbm, zeros_hbm, out_hbm, dstv, onesv, deg, sem):
    cid = lax.axis_index("c")
    sid = lax.axis_index("s")
    wid = cid * NS + sid
    r0 = sid * ROWS_PER_TILE
    pltpu.sync_copy(zeros_hbm, deg.at[pl.ds(r0, ROWS_PER_TILE)])
    pltpu.sync_copy(ones_hbm, onesv)
    pltpu.sync_copy(dst_hbm.at[wid], dstv)
    plsc.subcore_barrier()

    def d_desc(j):
        return pltpu.make_async_copy(onesv, deg.at[dstv.at[j]], sem)

    def body(j, carry):
        d_desc(j).start(add=True)

        @pl.when(j >= DEG_FD)
        def _():
            d_desc(j - DEG_FD).wait()

        return carry

    lax.fori_loop(0, NCH, body, 0)
    for k in range(DEG_FD):
        d_desc(NCH - DEG_FD + k).wait()
    plsc.subcore_barrier()
    pltpu.sync_copy(deg.at[pl.ds(r0, ROWS_PER_TILE)],
                    out_hbm.at[pl.ds(cid * NP + r0, ROWS_PER_TILE)])


ECONV = 80
CNCH = EPT_PAD // ECONV


@functools.partial(
    pl.kernel,
    out_type=jax.ShapeDtypeStruct((NC * NP, D), jnp.float32),
    mesh=_MESH,
    scratch_types=[
        pltpu.VMEM((ECONV,), jnp.int32),
        pltpu.VMEM((ECONV,), jnp.int32),
        pltpu.VMEM((ECONV, D), jnp.float32),
        pltpu.VMEM_SHARED((NP, D), jnp.float32),
        pltpu.SemaphoreType.DMA,
    ],
)
def _conv_sc(h_hbm, src_hbm, dst_hbm, out_hbm, sbuf, dbuf, rows, acc, sem):
    cid = lax.axis_index("c")
    sid = lax.axis_index("s")
    wid = cid * NS + sid
    r0 = sid * ROWS_PER_TILE
    pltpu.sync_copy(h_hbm.at[pl.ds(r0, ROWS_PER_TILE)],
                    acc.at[pl.ds(r0, ROWS_PER_TILE)])
    plsc.subcore_barrier()

    def body(j, carry):
        pltpu.sync_copy(src_hbm.at[wid, j], sbuf)
        pltpu.sync_copy(dst_hbm.at[wid, j], dbuf)
        pltpu.async_copy(h_hbm.at[sbuf], rows, sem).wait()
        pltpu.sync_copy(rows, acc.at[dbuf], add=True)
        return carry

    lax.fori_loop(0, CNCH, body, 0)
    plsc.subcore_barrier()
    pltpu.sync_copy(acc.at[pl.ds(r0, ROWS_PER_TILE)],
                    out_hbm.at[pl.ds(cid * NP + r0, ROWS_PER_TILE)])


@functools.partial(
    pl.kernel,
    out_type=jax.ShapeDtypeStruct((2 * Q, D), jnp.float32),
    mesh=_MESH,
    scratch_types=[
        pltpu.VMEM((QNCH, QCHUNK), jnp.int32),
    ]
    + [pltpu.VMEM((QCHUNK, D), jnp.float32)] * NBUF
    + [pltpu.SemaphoreType.DMA] * (2 * NBUF),
)
def _link_sc(x_hbm, q_hbm, out_hbm, qv, *bufs):
    rows = bufs[:NBUF]
    gsem = bufs[NBUF:2 * NBUF]
    ssem = bufs[2 * NBUF:]
    cid = lax.axis_index("c")
    sid = lax.axis_index("s")
    wid = cid * NS + sid
    pltpu.sync_copy(q_hbm.at[wid], qv)

    half = QNCH // 2

    def out_off(j):
        sel = j // half
        return sel * Q + wid * QPT + (j - sel * half) * QCHUNK

    def g_desc(b, j):
        return pltpu.make_async_copy(x_hbm.at[qv.at[j]], rows[b], gsem[b])

    def s_desc(b, j):
        return pltpu.make_async_copy(
            rows[b], out_hbm.at[pl.ds(out_off(j), QCHUNK)], ssem[b])

    for b in range(NBUF):
        g_desc(b, b).start()

    def outer(o, carry):
        j0 = o * NBUF
        for b in range(NBUF):
            g_desc(b, j0 + b).wait()
            s_desc(b, j0 + b).start()
        for b in range(NBUF):
            s_desc(b, j0 + b).wait()
            nj = j0 + NBUF + b

            @pl.when(nj < QNCH)
            def _():
                g_desc(b, nj).start()

        return carry

    lax.fori_loop(0, QNCH // NBUF, outer, 0)


_RB = 1024


def _dis_body(degp_ref, dis_ref):
    d = degp_ref[...]
    deg = d[0:NP, 0:1] + d[NP:2 * NP, 0:1] + 1.0
    dis_ref[...] = lax.rsqrt(deg)


def _dis_tc(degp):
    return pl.pallas_call(
        _dis_body,
        grid=(1,),
        in_specs=[pl.BlockSpec((NC * NP, DEGW), lambda i: (0, 0))],
        out_specs=pl.BlockSpec((NP, 1), lambda i: (0, 0)),
        out_shape=jax.ShapeDtypeStruct((NP, 1), jnp.float32),
    )(degp)


def _prep_body(dis_ref, emb_ref, w1_ref, h_ref):
    h_ref[...] = jnp.dot(emb_ref[...] * dis_ref[...], w1_ref[...],
                         preferred_element_type=jnp.float32)


def _prep_tc(dis, emb, w1):
    return pl.pallas_call(
        _prep_body,
        grid=(NP // _RB,),
        in_specs=[
            pl.BlockSpec((_RB, 1), lambda i: (i, 0)),
            pl.BlockSpec((_RB, D), lambda i: (i, 0)),
            pl.BlockSpec((D, D), lambda i: (0, 0)),
        ],
        out_specs=pl.BlockSpec((_RB, D), lambda i: (i, 0)),
        out_shape=jax.ShapeDtypeStruct((NP, D), jnp.float32),
    )(dis, emb, w1)


def _mid_body(acca_ref, accb_ref, hp_ref, dis_ref, b_ref, w_ref, out_ref):
    s = acca_ref[...] + accb_ref[...] - hp_ref[...]
    x1 = jnp.maximum(dis_ref[...] * s + b_ref[...], 0.0)
    out_ref[...] = jnp.dot(x1 * dis_ref[...], w_ref[...],
                           preferred_element_type=jnp.float32)


def _mid_tc(acc, hp, dis, b_row, w2):
    return pl.pallas_call(
        _mid_body,
        grid=(NP // _RB,),
        in_specs=[
            pl.BlockSpec((_RB, D), lambda i: (i, 0)),
            pl.BlockSpec((_RB, D), lambda i: (i + NP // _RB, 0)),
            pl.BlockSpec((_RB, D), lambda i: (i, 0)),
            pl.BlockSpec((_RB, 1), lambda i: (i, 0)),
            pl.BlockSpec((1, D), lambda i: (0, 0)),
            pl.BlockSpec((D, D), lambda i: (0, 0)),
        ],
        out_specs=pl.BlockSpec((_RB, D), lambda i: (i, 0)),
        out_shape=jax.ShapeDtypeStruct((NP, D), jnp.float32),
    )(acc, acc, hp, dis, b_row, w2)


def _final_body(acca_ref, accb_ref, hp_ref, dis_ref, b_ref, out_ref):
    s = acca_ref[...] + accb_ref[...] - hp_ref[...]
    out_ref[...] = dis_ref[...] * s + b_ref[...]


def _final_tc(acc, hp, dis, b_row):
    return pl.pallas_call(
        _final_body,
        grid=(NP // _RB,),
        in_specs=[
            pl.BlockSpec((_RB, D), lambda i: (i, 0)),
            pl.BlockSpec((_RB, D), lambda i: (i + NP // _RB, 0)),
            pl.BlockSpec((_RB, D), lambda i: (i, 0)),
            pl.BlockSpec((_RB, 1), lambda i: (i, 0)),
            pl.BlockSpec((1, D), lambda i: (0, 0)),
        ],
        out_specs=pl.BlockSpec((_RB, D), lambda i: (i, 0)),
        out_shape=jax.ShapeDtypeStruct((NP, D), jnp.float32),
    )(acc, acc, hp, dis, b_row)


_QB = 2048


def _pred_body(ga_ref, gb_ref, wp1_ref, bp1_ref, wp2_ref, bp2_ref, out_ref):
    h = ga_ref[...] * gb_ref[...]
    h = jnp.maximum(
        jnp.dot(h, wp1_ref[...], preferred_element_type=jnp.float32)
        + bp1_ref[...], 0.0)
    z = jnp.dot(h, wp2_ref[...], preferred_element_type=jnp.float32) \
        + bp2_ref[...]
    out_ref[...] = jax.nn.sigmoid(z)


def _pred_tc(gab, wp1, bp1_row, wp2, bp2_row):
    return pl.pallas_call(
        _pred_body,
        grid=(Q // _QB,),
        in_specs=[
            pl.BlockSpec((_QB, D), lambda i: (i, 0)),
            pl.BlockSpec((_QB, D), lambda i: (i + Q // _QB, 0)),
            pl.BlockSpec((D, D), lambda i: (0, 0)),
            pl.BlockSpec((1, D), lambda i: (0, 0)),
            pl.BlockSpec((D, 1), lambda i: (0, 0)),
            pl.BlockSpec((1, 1), lambda i: (0, 0)),
        ],
        out_specs=pl.BlockSpec((_QB, 1), lambda i: (i, 0)),
        out_shape=jax.ShapeDtypeStruct((Q, 1), jnp.float32),
    )(gab, gab, wp1, bp1_row, wp2, bp2_row)


def kernel(edge_index, edges, emb, W1, b1, W2, b2, Wp1, bp1, Wp2, bp2):
    src = edge_index[0].astype(jnp.int32)
    dst = edge_index[1].astype(jnp.int32)
    pad_rows = jnp.broadcast_to(
        jnp.arange(N, NP, dtype=jnp.int32)[None, :], (NW, EPT_PAD - EPT))
    src3 = jnp.concatenate([src.reshape(NW, EPT), pad_rows], axis=1) \
        .reshape(NW, NCH, ECHUNK)
    dst3 = jnp.concatenate([dst.reshape(NW, EPT), pad_rows], axis=1) \
        .reshape(NW, NCH, ECHUNK)
    srcc = src3.reshape(NW, CNCH, ECONV)
    dstc = dst3.reshape(NW, CNCH, ECONV)
    q0 = edges[0].astype(jnp.int32).reshape(NW, QNCH // 2, QCHUNK)
    q1 = edges[1].astype(jnp.int32).reshape(NW, QNCH // 2, QCHUNK)
    qall = jnp.concatenate([q0, q1], axis=1)

    emb_p = jnp.pad(emb, ((0, NP - N), (0, 0)))
    degp = _deg_sc(dst3,
                   jnp.ones((ECHUNK, DEGW), jnp.float32),
                   jnp.zeros((ROWS_PER_TILE, DEGW), jnp.float32))
    dis = _dis_tc(degp)
    h1p = _prep_tc(dis, emb_p, W1)
    acc1 = _conv_sc(h1p, srcc, dstc)
    h2p = _mid_tc(acc1, h1p, dis, b1.reshape(1, D), W2)
    acc2 = _conv_sc(h2p, srcc, dstc)
    x2 = _final_tc(acc2, h2p, dis, b2.reshape(1, D))
    gab = _link_sc(x2, qall)
    out = _pred_tc(gab, Wp1, bp1.reshape(1, D), Wp2, bp2.reshape(1, 1))
    return out[:, 0]


# speedup vs baseline: 2.6951x; 1.7044x over previous
import functools

import jax
import jax.numpy as jnp
from jax import lax
from jax.experimental import pallas as pl
from jax.experimental.pallas import tpu as pltpu
from jax.experimental.pallas import tpu_sc as plsc

N = 10000
E = 320000
Q = 65536
D = 128
NP = 10240

NC = 2
NS = 16
NW = NC * NS

ECHUNK = 64
EPT = E // NW
EPT_PAD = 10240
NCH = EPT_PAD // ECHUNK
NBUF = 2
NG = NCH // NBUF

QCHUNK = 128
QNCH = 2 * (Q // NW) // QCHUNK
QPT = Q // NW

ROWS_PER_TILE = NP // NS

_MESH = plsc.VectorSubcoreMesh(core_axis_name="c", subcore_axis_name="s")


DEGW = 128
DEG_FD = 8


@functools.partial(
    pl.kernel,
    out_type=jax.ShapeDtypeStruct((NC * NP, DEGW), jnp.float32),
    mesh=_MESH,
    scratch_types=[
        pltpu.VMEM((NCH, ECHUNK), jnp.int32),
        pltpu.VMEM((ECHUNK, DEGW), jnp.float32),
        pltpu.VMEM_SHARED((NP, DEGW), jnp.float32),
        pltpu.SemaphoreType.DMA,
    ],
)
def _deg_sc(dst_hbm, ones_hbm, zeros_hbm, out_hbm, dstv, onesv, deg, sem):
    cid = lax.axis_index("c")
    sid = lax.axis_index("s")
    wid = cid * NS + sid
    r0 = sid * ROWS_PER_TILE
    pltpu.sync_copy(zeros_hbm, deg.at[pl.ds(r0, ROWS_PER_TILE)])
    pltpu.sync_copy(ones_hbm, onesv)
    pltpu.sync_copy(dst_hbm.at[wid], dstv)
    plsc.subcore_barrier()

    def d_desc(j):
        return pltpu.make_async_copy(onesv, deg.at[dstv.at[j]], sem)

    def body(j, carry):
        d_desc(j).start(add=True)

        @pl.when(j >= DEG_FD)
        def _():
            d_desc(j - DEG_FD).wait()

        return carry

    lax.fori_loop(0, NCH, body, 0)
    for k in range(DEG_FD):
        d_desc(NCH - DEG_FD + k).wait()
    plsc.subcore_barrier()
    pltpu.sync_copy(deg.at[pl.ds(r0, ROWS_PER_TILE)],
                    out_hbm.at[pl.ds(cid * NP + r0, ROWS_PER_TILE)])


ECONV = 128
CNCH = EPT_PAD // ECONV


@functools.partial(
    pl.kernel,
    out_type=jax.ShapeDtypeStruct((NC * NP, D), jnp.float32),
    mesh=_MESH,
    scratch_types=[
        pltpu.VMEM((CNCH, ECONV), jnp.int32),
        pltpu.VMEM_SHARED((NP, D), jnp.float32),
    ]
    + [pltpu.VMEM((ECONV,), jnp.int32)] * NBUF
    + [pltpu.VMEM((ECONV, D), jnp.float32)] * NBUF
    + [pltpu.SemaphoreType.DMA] * (3 * NBUF),
)
def _conv_sc(h_hbm, src_hbm, dst_hbm, out_hbm, dstv, acc, *bufs):
    sidx = bufs[:NBUF]
    rows = bufs[NBUF:2 * NBUF]
    isem = bufs[2 * NBUF:3 * NBUF]
    gsem = bufs[3 * NBUF:4 * NBUF]
    ssem = bufs[4 * NBUF:]
    cid = lax.axis_index("c")
    sid = lax.axis_index("s")
    wid = cid * NS + sid
    r0 = sid * ROWS_PER_TILE

    def i_desc(b, j):
        return pltpu.make_async_copy(src_hbm.at[wid, j], sidx[b], isem[b])

    def g_desc(b, j):
        return pltpu.make_async_copy(h_hbm.at[sidx[b]], rows[b], gsem[b])

    def s_desc(b, j):
        return pltpu.make_async_copy(rows[b], acc.at[dstv.at[j]], ssem[b])

    i_desc(0, 0).start()
    i_desc(1, 1).start()
    pltpu.sync_copy(dst_hbm.at[wid], dstv)

    def initb(p, carry):
        pltpu.sync_copy(h_hbm.at[pl.ds(r0 + p * ECONV, ECONV)], rows[0])
        pltpu.sync_copy(rows[0], acc.at[pl.ds(r0 + p * ECONV, ECONV)])
        return carry

    lax.fori_loop(0, ROWS_PER_TILE // ECONV, initb, 0)
    plsc.subcore_barrier()

    i_desc(0, 0).wait()
    g_desc(0, 0).start()

    def step(j, b, b1):
        g_desc(b, j).wait()
        s_desc(b, j).start(add=True)

        @pl.when(j >= 1)
        def _():
            s_desc(b1, j - 1).wait()

        @pl.when(j + 1 < CNCH)
        def _():
            i_desc(b1, j + 1).wait()
            g_desc(b1, j + 1).start()

        @pl.when(j + 2 < CNCH)
        def _():
            i_desc(b, j + 2).start()

    def outer(o, carry):
        j0 = o * 2
        step(j0, 0, 1)
        step(j0 + 1, 1, 0)
        return carry

    lax.fori_loop(0, CNCH // 2, outer, 0)
    s_desc((CNCH - 1) % 2, CNCH - 1).wait()
    plsc.subcore_barrier()

    def outb(p, carry):
        pltpu.sync_copy(acc.at[pl.ds(r0 + p * ECONV, ECONV)], rows[0])
        pltpu.sync_copy(rows[0],
                        out_hbm.at[pl.ds(cid * NP + r0 + p * ECONV, ECONV)])
        return carry

    lax.fori_loop(0, ROWS_PER_TILE // ECONV, outb, 0)


@functools.partial(
    pl.kernel,
    out_type=jax.ShapeDtypeStruct((2 * Q, D), jnp.float32),
    mesh=_MESH,
    scratch_types=[
        pltpu.VMEM((QNCH, QCHUNK), jnp.int32),
    ]
    + [pltpu.VMEM((QCHUNK, D), jnp.float32)] * NBUF
    + [pltpu.SemaphoreType.DMA] * (2 * NBUF),
)
def _link_sc(x_hbm, q_hbm, out_hbm, qv, *bufs):
    rows = bufs[:NBUF]
    gsem = bufs[NBUF:2 * NBUF]
    ssem = bufs[2 * NBUF:]
    cid = lax.axis_index("c")
    sid = lax.axis_index("s")
    wid = cid * NS + sid
    pltpu.sync_copy(q_hbm.at[wid], qv)

    half = QNCH // 2

    def out_off(j):
        sel = j // half
        return sel * Q + wid * QPT + (j - sel * half) * QCHUNK

    def g_desc(b, j):
        return pltpu.make_async_copy(x_hbm.at[qv.at[j]], rows[b], gsem[b])

    def s_desc(b, j):
        return pltpu.make_async_copy(
            rows[b], out_hbm.at[pl.ds(out_off(j), QCHUNK)], ssem[b])

    for b in range(NBUF):
        g_desc(b, b).start()

    def outer(o, carry):
        j0 = o * NBUF
        for b in range(NBUF):
            g_desc(b, j0 + b).wait()
            s_desc(b, j0 + b).start()
        for b in range(NBUF):
            s_desc(b, j0 + b).wait()
            nj = j0 + NBUF + b

            @pl.when(nj < QNCH)
            def _():
                g_desc(b, nj).start()

        return carry

    lax.fori_loop(0, QNCH // NBUF, outer, 0)


_RB = 1024


def _dis_body(degp_ref, dis_ref):
    d = degp_ref[...]
    deg = d[0:NP, 0:1] + d[NP:2 * NP, 0:1] + 1.0
    dis_ref[...] = lax.rsqrt(deg)


def _dis_tc(degp):
    return pl.pallas_call(
        _dis_body,
        grid=(1,),
        in_specs=[pl.BlockSpec((NC * NP, DEGW), lambda i: (0, 0))],
        out_specs=pl.BlockSpec((NP, 1), lambda i: (0, 0)),
        out_shape=jax.ShapeDtypeStruct((NP, 1), jnp.float32),
    )(degp)


def _prep_body(dis_ref, emb_ref, w1_ref, h_ref):
    h_ref[...] = jnp.dot(emb_ref[...] * dis_ref[...], w1_ref[...],
                         preferred_element_type=jnp.float32)


def _prep_tc(dis, emb, w1):
    return pl.pallas_call(
        _prep_body,
        grid=(NP // _RB,),
        in_specs=[
            pl.BlockSpec((_RB, 1), lambda i: (i, 0)),
            pl.BlockSpec((_RB, D), lambda i: (i, 0)),
            pl.BlockSpec((D, D), lambda i: (0, 0)),
        ],
        out_specs=pl.BlockSpec((_RB, D), lambda i: (i, 0)),
        out_shape=jax.ShapeDtypeStruct((NP, D), jnp.float32),
    )(dis, emb, w1)


def _mid_body(acca_ref, accb_ref, hp_ref, dis_ref, b_ref, w_ref, out_ref):
    s = acca_ref[...] + accb_ref[...] - hp_ref[...]
    x1 = jnp.maximum(dis_ref[...] * s + b_ref[...], 0.0)
    out_ref[...] = jnp.dot(x1 * dis_ref[...], w_ref[...],
                           preferred_element_type=jnp.float32)


def _mid_tc(acc, hp, dis, b_row, w2):
    return pl.pallas_call(
        _mid_body,
        grid=(NP // _RB,),
        in_specs=[
            pl.BlockSpec((_RB, D), lambda i: (i, 0)),
            pl.BlockSpec((_RB, D), lambda i: (i + NP // _RB, 0)),
            pl.BlockSpec((_RB, D), lambda i: (i, 0)),
            pl.BlockSpec((_RB, 1), lambda i: (i, 0)),
            pl.BlockSpec((1, D), lambda i: (0, 0)),
            pl.BlockSpec((D, D), lambda i: (0, 0)),
        ],
        out_specs=pl.BlockSpec((_RB, D), lambda i: (i, 0)),
        out_shape=jax.ShapeDtypeStruct((NP, D), jnp.float32),
    )(acc, acc, hp, dis, b_row, w2)


def _final_body(acca_ref, accb_ref, hp_ref, dis_ref, b_ref, out_ref):
    s = acca_ref[...] + accb_ref[...] - hp_ref[...]
    out_ref[...] = dis_ref[...] * s + b_ref[...]


def _final_tc(acc, hp, dis, b_row):
    return pl.pallas_call(
        _final_body,
        grid=(NP // _RB,),
        in_specs=[
            pl.BlockSpec((_RB, D), lambda i: (i, 0)),
            pl.BlockSpec((_RB, D), lambda i: (i + NP // _RB, 0)),
            pl.BlockSpec((_RB, D), lambda i: (i, 0)),
            pl.BlockSpec((_RB, 1), lambda i: (i, 0)),
            pl.BlockSpec((1, D), lambda i: (0, 0)),
        ],
        out_specs=pl.BlockSpec((_RB, D), lambda i: (i, 0)),
        out_shape=jax.ShapeDtypeStruct((NP, D), jnp.float32),
    )(acc, acc, hp, dis, b_row)


_QB = 2048


def _pred_body(ga_ref, gb_ref, wp1_ref, bp1_ref, wp2_ref, bp2_ref, out_ref):
    h = ga_ref[...] * gb_ref[...]
    h = jnp.maximum(
        jnp.dot(h, wp1_ref[...], preferred_element_type=jnp.float32)
        + bp1_ref[...], 0.0)
    z = jnp.dot(h, wp2_ref[...], preferred_element_type=jnp.float32) \
        + bp2_ref[...]
    out_ref[...] = jax.nn.sigmoid(z)


def _pred_tc(gab, wp1, bp1_row, wp2, bp2_row):
    return pl.pallas_call(
        _pred_body,
        grid=(Q // _QB,),
        in_specs=[
            pl.BlockSpec((_QB, D), lambda i: (i, 0)),
            pl.BlockSpec((_QB, D), lambda i: (i + Q // _QB, 0)),
            pl.BlockSpec((D, D), lambda i: (0, 0)),
            pl.BlockSpec((1, D), lambda i: (0, 0)),
            pl.BlockSpec((D, 1), lambda i: (0, 0)),
            pl.BlockSpec((1, 1), lambda i: (0, 0)),
        ],
        out_specs=pl.BlockSpec((_QB, 1), lambda i: (i, 0)),
        out_shape=jax.ShapeDtypeStruct((Q, 1), jnp.float32),
    )(gab, gab, wp1, bp1_row, wp2, bp2_row)


def kernel(edge_index, edges, emb, W1, b1, W2, b2, Wp1, bp1, Wp2, bp2):
    src = edge_index[0].astype(jnp.int32)
    dst = edge_index[1].astype(jnp.int32)
    pad_rows = jnp.broadcast_to(
        jnp.arange(N, NP, dtype=jnp.int32)[None, :], (NW, EPT_PAD - EPT))
    src3 = jnp.concatenate([src.reshape(NW, EPT), pad_rows], axis=1) \
        .reshape(NW, NCH, ECHUNK)
    dst3 = jnp.concatenate([dst.reshape(NW, EPT), pad_rows], axis=1) \
        .reshape(NW, NCH, ECHUNK)
    srcc = src3.reshape(NW, CNCH, ECONV)
    dstc = dst3.reshape(NW, CNCH, ECONV)
    q0 = edges[0].astype(jnp.int32).reshape(NW, QNCH // 2, QCHUNK)
    q1 = edges[1].astype(jnp.int32).reshape(NW, QNCH // 2, QCHUNK)
    qall = jnp.concatenate([q0, q1], axis=1)

    emb_p = jnp.pad(emb, ((0, NP - N), (0, 0)))
    degp = _deg_sc(dst3,
                   jnp.ones((ECHUNK, DEGW), jnp.float32),
                   jnp.zeros((ROWS_PER_TILE, DEGW), jnp.float32))
    dis = _dis_tc(degp)
    h1p = _prep_tc(dis, emb_p, W1)
    acc1 = _conv_sc(h1p, srcc, dstc)
    h2p = _mid_tc(acc1, h1p, dis, b1.reshape(1, D), W2)
    acc2 = _conv_sc(h2p, srcc, dstc)
    x2 = _final_tc(acc2, h2p, dis, b2.reshape(1, D))
    gab = _link_sc(x2, qall)
    out = _pred_tc(gab, Wp1, bp1.reshape(1, D), Wp2, bp2.reshape(1, 1))
    return out[:, 0]


# speedup vs baseline: 2.7273x; 1.0119x over previous
import functools

import jax
import jax.numpy as jnp
from jax import lax
from jax.experimental import pallas as pl
from jax.experimental.pallas import tpu as pltpu
from jax.experimental.pallas import tpu_sc as plsc

N = 10000
E = 320000
Q = 65536
D = 128
NP = 10240

NC = 2
NS = 16
NW = NC * NS

ECHUNK = 64
EPT = E // NW
EPT_PAD = 10240
NCH = EPT_PAD // ECHUNK
NBUF = 2
NG = NCH // NBUF

QCHUNK = 128
QNCH = 2 * (Q // NW) // QCHUNK
QPT = Q // NW

ROWS_PER_TILE = NP // NS

_MESH = plsc.VectorSubcoreMesh(core_axis_name="c", subcore_axis_name="s")


DEGW = 128
DEG_FD = 8


@functools.partial(
    pl.kernel,
    out_type=jax.ShapeDtypeStruct((NC * NP, DEGW), jnp.float32),
    mesh=_MESH,
    scratch_types=[
        pltpu.VMEM((NCH, ECHUNK), jnp.int32),
        pltpu.VMEM((ECHUNK, DEGW), jnp.float32),
        pltpu.VMEM_SHARED((NP, DEGW), jnp.float32),
        pltpu.SemaphoreType.DMA,
    ],
)
def _deg_sc(dst_hbm, ones_hbm, zeros_hbm, out_hbm, dstv, onesv, deg, sem):
    cid = lax.axis_index("c")
    sid = lax.axis_index("s")
    wid = cid * NS + sid
    r0 = sid * ROWS_PER_TILE
    pltpu.sync_copy(zeros_hbm, deg.at[pl.ds(r0, ROWS_PER_TILE)])
    pltpu.sync_copy(ones_hbm, onesv)
    pltpu.sync_copy(dst_hbm.at[wid], dstv)
    plsc.subcore_barrier()

    def d_desc(j):
        return pltpu.make_async_copy(onesv, deg.at[dstv.at[j]], sem)

    def body(j, carry):
        d_desc(j).start(add=True)

        @pl.when(j >= DEG_FD)
        def _():
            d_desc(j - DEG_FD).wait()

        return carry

    lax.fori_loop(0, NCH, body, 0)
    for k in range(DEG_FD):
        d_desc(NCH - DEG_FD + k).wait()
    plsc.subcore_barrier()
    pltpu.sync_copy(deg.at[pl.ds(r0, ROWS_PER_TILE)],
                    out_hbm.at[pl.ds(cid * NP + r0, ROWS_PER_TILE)])


ECONV = 128
CNCH = EPT_PAD // ECONV


@functools.partial(
    pl.kernel,
    out_type=jax.ShapeDtypeStruct((NC * NP, D), jnp.float32),
    mesh=_MESH,
    scratch_types=[
        pltpu.VMEM((CNCH, ECONV), jnp.int32),
        pltpu.VMEM_SHARED((NP, D), jnp.float32),
    ]
    + [pltpu.VMEM((ECONV,), jnp.int32)] * NBUF
    + [pltpu.VMEM((ECONV, D), jnp.float32)] * NBUF
    + [pltpu.SemaphoreType.DMA] * (3 * NBUF),
)
def _conv_sc(h_hbm, src_hbm, dst_hbm, out_hbm, dstv, acc, *bufs):
    sidx = bufs[:NBUF]
    rows = bufs[NBUF:2 * NBUF]
    isem = bufs[2 * NBUF:3 * NBUF]
    gsem = bufs[3 * NBUF:4 * NBUF]
    ssem = bufs[4 * NBUF:]
    cid = lax.axis_index("c")
    sid = lax.axis_index("s")
    wid = cid * NS + sid
    r0 = sid * ROWS_PER_TILE

    def i_desc(b, j):
        return pltpu.make_async_copy(src_hbm.at[wid, j], sidx[b], isem[b])

    def g_desc(b, j):
        return pltpu.make_async_copy(h_hbm.at[sidx[b]], rows[b], gsem[b])

    def s_desc(b, j):
        return pltpu.make_async_copy(rows[b], acc.at[dstv.at[j]], ssem[b])

    i_desc(0, 0).start()
    i_desc(1, 1).start()
    pltpu.sync_copy(dst_hbm.at[wid], dstv)

    def initb(p, carry):
        pltpu.sync_copy(h_hbm.at[pl.ds(r0 + p * ECONV, ECONV)], rows[0])
        pltpu.sync_copy(rows[0], acc.at[pl.ds(r0 + p * ECONV, ECONV)])
        return carry

    lax.fori_loop(0, ROWS_PER_TILE // ECONV, initb, 0)
    plsc.subcore_barrier()

    i_desc(0, 0).wait()
    g_desc(0, 0).start()

    def step(j, b, b1):
        g_desc(b, j).wait()
        s_desc(b, j).start(add=True)

        @pl.when(j >= 1)
        def _():
            s_desc(b1, j - 1).wait()

        @pl.when(j + 1 < CNCH)
        def _():
            i_desc(b1, j + 1).wait()
            g_desc(b1, j + 1).start()

        @pl.when(j + 2 < CNCH)
        def _():
            i_desc(b, j + 2).start()

    def outer(o, carry):
        j0 = o * 2
        step(j0, 0, 1)
        step(j0 + 1, 1, 0)
        return carry

    lax.fori_loop(0, CNCH // 2, outer, 0)
    s_desc((CNCH - 1) % 2, CNCH - 1).wait()
    plsc.subcore_barrier()

    def outb(p, carry):
        pltpu.sync_copy(acc.at[pl.ds(r0 + p * ECONV, ECONV)], rows[0])
        pltpu.sync_copy(rows[0],
                        out_hbm.at[pl.ds(cid * NP + r0 + p * ECONV, ECONV)])
        return carry

    lax.fori_loop(0, ROWS_PER_TILE // ECONV, outb, 0)


@functools.partial(
    pl.kernel,
    out_type=jax.ShapeDtypeStruct((2 * Q, D), jnp.float32),
    mesh=_MESH,
    scratch_types=[
        pltpu.VMEM((QNCH, QCHUNK), jnp.int32),
    ]
    + [pltpu.VMEM((QCHUNK, D), jnp.float32)] * NBUF
    + [pltpu.SemaphoreType.DMA] * (2 * NBUF),
)
def _link_sc(x_hbm, q_hbm, out_hbm, qv, *bufs):
    rows = bufs[:NBUF]
    gsem = bufs[NBUF:2 * NBUF]
    ssem = bufs[2 * NBUF:]
    cid = lax.axis_index("c")
    sid = lax.axis_index("s")
    wid = cid * NS + sid
    pltpu.sync_copy(q_hbm.at[wid], qv)

    half = QNCH // 2

    def out_off(j):
        sel = j // half
        return sel * Q + wid * QPT + (j - sel * half) * QCHUNK

    def g_desc(b, j):
        return pltpu.make_async_copy(x_hbm.at[qv.at[j]], rows[b], gsem[b])

    def s_desc(b, j):
        return pltpu.make_async_copy(
            rows[b], out_hbm.at[pl.ds(out_off(j), QCHUNK)], ssem[b])

    for b in range(NBUF):
        g_desc(b, b).start()

    def outer(o, carry):
        j0 = o * NBUF
        for b in range(NBUF):
            g_desc(b, j0 + b).wait()
            s_desc(b, j0 + b).start()
        for b in range(NBUF):
            s_desc(b, j0 + b).wait()
            nj = j0 + NBUF + b

            @pl.when(nj < QNCH)
            def _():
                g_desc(b, nj).start()

        return carry

    lax.fori_loop(0, QNCH // NBUF, outer, 0)


_RB = 1024


def _prep_body(degp0_ref, degp1_ref, emb_ref, w1_ref, h_ref, dis_ref):
    deg = degp0_ref[:, 0:1] + degp1_ref[:, 0:1] + 1.0
    dis = lax.rsqrt(deg)
    dis_ref[...] = dis
    h_ref[...] = jnp.dot(emb_ref[...] * dis, w1_ref[...],
                         preferred_element_type=jnp.float32)


def _prep_tc(degp, emb, w1):
    return pl.pallas_call(
        _prep_body,
        grid=(NP // _RB,),
        in_specs=[
            pl.BlockSpec((_RB, DEGW), lambda i: (i, 0)),
            pl.BlockSpec((_RB, DEGW), lambda i: (i + NP // _RB, 0)),
            pl.BlockSpec((_RB, D), lambda i: (i, 0)),
            pl.BlockSpec((D, D), lambda i: (0, 0)),
        ],
        out_specs=[
            pl.BlockSpec((_RB, D), lambda i: (i, 0)),
            pl.BlockSpec((_RB, 1), lambda i: (i, 0)),
        ],
        out_shape=[
            jax.ShapeDtypeStruct((NP, D), jnp.float32),
            jax.ShapeDtypeStruct((NP, 1), jnp.float32),
        ],
    )(degp, degp, emb, w1)


def _mid_body(acca_ref, accb_ref, hp_ref, dis_ref, b_ref, w_ref, out_ref):
    s = acca_ref[...] + accb_ref[...] - hp_ref[...]
    x1 = jnp.maximum(dis_ref[...] * s + b_ref[...], 0.0)
    out_ref[...] = jnp.dot(x1 * dis_ref[...], w_ref[...],
                           preferred_element_type=jnp.float32)


def _mid_tc(acc, hp, dis, b_row, w2):
    return pl.pallas_call(
        _mid_body,
        grid=(NP // _RB,),
        in_specs=[
            pl.BlockSpec((_RB, D), lambda i: (i, 0)),
            pl.BlockSpec((_RB, D), lambda i: (i + NP // _RB, 0)),
            pl.BlockSpec((_RB, D), lambda i: (i, 0)),
            pl.BlockSpec((_RB, 1), lambda i: (i, 0)),
            pl.BlockSpec((1, D), lambda i: (0, 0)),
            pl.BlockSpec((D, D), lambda i: (0, 0)),
        ],
        out_specs=pl.BlockSpec((_RB, D), lambda i: (i, 0)),
        out_shape=jax.ShapeDtypeStruct((NP, D), jnp.float32),
    )(acc, acc, hp, dis, b_row, w2)


def _final_body(acca_ref, accb_ref, hp_ref, dis_ref, b_ref, out_ref):
    s = acca_ref[...] + accb_ref[...] - hp_ref[...]
    out_ref[...] = dis_ref[...] * s + b_ref[...]


def _final_tc(acc, hp, dis, b_row):
    return pl.pallas_call(
        _final_body,
        grid=(NP // _RB,),
        in_specs=[
            pl.BlockSpec((_RB, D), lambda i: (i, 0)),
            pl.BlockSpec((_RB, D), lambda i: (i + NP // _RB, 0)),
            pl.BlockSpec((_RB, D), lambda i: (i, 0)),
            pl.BlockSpec((_RB, 1), lambda i: (i, 0)),
            pl.BlockSpec((1, D), lambda i: (0, 0)),
        ],
        out_specs=pl.BlockSpec((_RB, D), lambda i: (i, 0)),
        out_shape=jax.ShapeDtypeStruct((NP, D), jnp.float32),
    )(acc, acc, hp, dis, b_row)


_QB = 2048


def _pred_body(ga_ref, gb_ref, wp1_ref, bp1_ref, wp2_ref, bp2_ref, out_ref):
    h = ga_ref[...] * gb_ref[...]
    h = jnp.maximum(
        jnp.dot(h, wp1_ref[...], preferred_element_type=jnp.float32)
        + bp1_ref[...], 0.0)
    z = jnp.dot(h, wp2_ref[...], preferred_element_type=jnp.float32) \
        + bp2_ref[...]
    out_ref[...] = jax.nn.sigmoid(z)


def _pred_tc(gab, wp1, bp1_row, wp2, bp2_row):
    return pl.pallas_call(
        _pred_body,
        grid=(Q // _QB,),
        in_specs=[
            pl.BlockSpec((_QB, D), lambda i: (i, 0)),
            pl.BlockSpec((_QB, D), lambda i: (i + Q // _QB, 0)),
            pl.BlockSpec((D, D), lambda i: (0, 0)),
            pl.BlockSpec((1, D), lambda i: (0, 0)),
            pl.BlockSpec((D, 1), lambda i: (0, 0)),
            pl.BlockSpec((1, 1), lambda i: (0, 0)),
        ],
        out_specs=pl.BlockSpec((_QB, 1), lambda i: (i, 0)),
        out_shape=jax.ShapeDtypeStruct((Q, 1), jnp.float32),
    )(gab, gab, wp1, bp1_row, wp2, bp2_row)


def kernel(edge_index, edges, emb, W1, b1, W2, b2, Wp1, bp1, Wp2, bp2):
    src = edge_index[0].astype(jnp.int32)
    dst = edge_index[1].astype(jnp.int32)
    pad_rows = jnp.broadcast_to(
        jnp.arange(N, NP, dtype=jnp.int32)[None, :], (NW, EPT_PAD - EPT))
    src3 = jnp.concatenate([src.reshape(NW, EPT), pad_rows], axis=1) \
        .reshape(NW, NCH, ECHUNK)
    dst3 = jnp.concatenate([dst.reshape(NW, EPT), pad_rows], axis=1) \
        .reshape(NW, NCH, ECHUNK)
    srcc = src3.reshape(NW, CNCH, ECONV)
    dstc = dst3.reshape(NW, CNCH, ECONV)
    q0 = edges[0].astype(jnp.int32).reshape(NW, QNCH // 2, QCHUNK)
    q1 = edges[1].astype(jnp.int32).reshape(NW, QNCH // 2, QCHUNK)
    qall = jnp.concatenate([q0, q1], axis=1)

    emb_p = jnp.pad(emb, ((0, NP - N), (0, 0)))
    degp = _deg_sc(dst3,
                   jnp.ones((ECHUNK, DEGW), jnp.float32),
                   jnp.zeros((ROWS_PER_TILE, DEGW), jnp.float32))
    h1p, dis = _prep_tc(degp, emb_p, W1)
    acc1 = _conv_sc(h1p, srcc, dstc)
    h2p = _mid_tc(acc1, h1p, dis, b1.reshape(1, D), W2)
    acc2 = _conv_sc(h2p, srcc, dstc)
    x2 = _final_tc(acc2, h2p, dis, b2.reshape(1, D))
    gab = _link_sc(x2, qall)
    out = _pred_tc(gab, Wp1, bp1.reshape(1, D), Wp2, bp2.reshape(1, 1))
    return out[:, 0]
